# baseline (speedup 1.0000x reference)
import jax
import jax.numpy as jnp
import numpy as np
from jax import lax
from jax.experimental import pallas as pl
from jax.experimental.pallas import tpu as pltpu
from jax.experimental.pallas import tpu_sc as plsc

_NUM_USERS = 1000000
_EMB = 128
_LAT = 64
_BATCH = 16384

_SEC = 4096
_MAJ = 4 * _SEC
_NBLK = -(-_NUM_USERS // _MAJ)
_ROWS = _NBLK * _SEC
_LS = _SEC.bit_length() - 1
_LM = _MAJ.bit_length() - 1


def _placement_mats():
    e = np.zeros((8, _LAT, 128), np.float32)
    for k in range(4):
        for w in range(32):
            e[2 * k, w, 32 * k + w] = 1.0
            e[2 * k + 1, 32 + w, 32 * k + w] = 1.0
    return jnp.asarray(e)


def _repack_body(x_ref, e_ref, out_ref):
    dn = (((0,), (0,)), ((), ()))
    acc1 = None
    acc2 = None
    for k in range(4):
        x = x_ref[:, k * _SEC:(k + 1) * _SEC]
        t1 = lax.dot_general(x, e_ref[2 * k], dn,
                             precision=lax.Precision.DEFAULT,
                             preferred_element_type=jnp.float32)
        t2 = lax.dot_general(x, e_ref[2 * k + 1], dn,
                             precision=lax.Precision.DEFAULT,
                             preferred_element_type=jnp.float32)
        acc1 = t1 if acc1 is None else acc1 + t1
        acc2 = t2 if acc2 is None else acc2 + t2
    word = lax.bitcast_convert_type(acc1, jnp.int32) | \
        lax.shift_right_logical(lax.bitcast_convert_type(acc2, jnp.int32), 16)
    out_ref[...] = lax.bitcast_convert_type(word, jnp.float32)


def _repack_table(table_t, emats):
    return pl.pallas_call(
        _repack_body,
        grid=(_NBLK,),
        in_specs=[
            pl.BlockSpec((_LAT, _MAJ), lambda i: (0, i)),
            pl.BlockSpec((8, _LAT, 128), lambda i: (0, 0, 0)),
        ],
        out_specs=pl.BlockSpec((_SEC, 128), lambda i: (i, 0)),
        out_shape=jax.ShapeDtypeStruct((_ROWS, 128), jnp.float32),
    )(table_t, emats)


_NC, _NS = 2, 16
_NW = _NC * _NS
_BPW = _BATCH // _NW
_CHUNK = 128
_NCH = _BPW // _CHUNK


def _gather_body(tp_hbm, idx_hbm, out_hbm, idx_v, pidx_v, g_v, sem):
    wid = lax.axis_index("s") * _NC + lax.axis_index("c")
    base = wid * _BPW
    pltpu.sync_copy(idx_hbm.at[pl.ds(base, _BPW)], idx_v)
    for j in range(_NCH):
        for g in range(_CHUNK // 16):
            s = pl.ds(j * _CHUNK + g * 16, 16)
            u = idx_v[s]
            r = lax.shift_left(lax.shift_right_logical(u, _LM), _LS) | \
                (u & (_SEC - 1))
            pidx_v[j, pl.ds(g * 16, 16)] = r
    copies = [
        pltpu.async_copy(tp_hbm.at[pidx_v.at[j]], g_v.at[j], sem)
        for j in range(_NCH)
    ]
    for j in range(_NCH):
        copies[j].wait()
        pltpu.sync_copy(g_v.at[j],
                        out_hbm.at[pl.ds(base + j * _CHUNK, _CHUNK)])


_sc_gather = pl.kernel(
    _gather_body,
    out_type=jax.ShapeDtypeStruct((_BATCH, 128), jnp.float32),
    mesh=plsc.VectorSubcoreMesh(core_axis_name="c", subcore_axis_name="s"),
    scratch_types=[
        pltpu.VMEM((_BPW,), jnp.int32),
        pltpu.VMEM((_NCH, _CHUNK), jnp.int32),
        pltpu.VMEM((_NCH, _CHUNK, 128), jnp.float32),
        pltpu.SemaphoreType.DMA,
    ],
)

_BLK = 2048


def _combine_body(emb_ref, wh_ref, wl_ref, bh_ref, bl_ref, g_ref, idx_ref,
                  out_ref):
    vh = jnp.dot(emb_ref[...], wh_ref[...],
                 preferred_element_type=jnp.float32,
                 precision=lax.Precision.DEFAULT) + bh_ref[...]
    vl = jnp.dot(emb_ref[...], wl_ref[...],
                 preferred_element_type=jnp.float32,
                 precision=lax.Precision.DEFAULT) + bl_ref[...]
    gbits = lax.bitcast_convert_type(g_ref[...], jnp.int32)
    u_hi = lax.bitcast_convert_type(gbits & jnp.int32(-65536), jnp.float32)
    u_lo = lax.bitcast_convert_type(gbits << 16, jnp.float32)
    idx = idx_ref[...]
    s = (idx & (_MAJ - 1)) >> _LS
    laneq = lax.broadcasted_iota(jnp.int32, (_BLK, 128), 1) >> 5
    m = jnp.where(laneq == s, 1.0, 0.0)
    prod = (u_hi * vh + u_lo * vl) * m
    ones = jnp.ones((1, 128), jnp.float32)
    srow = lax.dot_general(ones, prod, (((1,), (1,)), ((), ())),
                           precision=lax.Precision.HIGHEST,
                           preferred_element_type=jnp.float32)
    out_ref[...] = srow[None]


def _tc_combine(g_rows, idx2, item_emb, wh, wl, bh, bl):
    out = pl.pallas_call(
        _combine_body,
        grid=(_BATCH // _BLK,),
        in_specs=[
            pl.BlockSpec((_BLK, _EMB), lambda i: (i, 0)),
            pl.BlockSpec((_EMB, 128), lambda i: (0, 0)),
            pl.BlockSpec((_EMB, 128), lambda i: (0, 0)),
            pl.BlockSpec((1, 128), lambda i: (0, 0)),
            pl.BlockSpec((1, 128), lambda i: (0, 0)),
            pl.BlockSpec((_BLK, 128), lambda i: (i, 0)),
            pl.BlockSpec((_BLK, 1), lambda i: (i, 0)),
        ],
        out_specs=pl.BlockSpec((1, 1, _BLK), lambda i: (i, 0, 0)),
        out_shape=jax.ShapeDtypeStruct((_BATCH // _BLK, 1, _BLK), jnp.float32),
    )(item_emb, wh, wl, bh, bl, g_rows, idx2)
    return out.reshape(-1)


def kernel(user_idx, item_emb, user_table, W, b):
    idx = user_idx.astype(jnp.int32)
    table_packed = _repack_table(user_table.T, _placement_mats())
    g_rows = _sc_gather(table_packed, idx)
    idx2 = idx.reshape(_BATCH, 1)
    wh = jnp.tile(W[:, :32], (1, 4))
    wl = jnp.tile(W[:, 32:], (1, 4))
    bh = jnp.tile(b[:32], 4).reshape(1, 128)
    bl = jnp.tile(b[32:], 4).reshape(1, 128)
    return _tc_combine(g_rows, idx2, item_emb, wh, wl, bh, bl)


# speedup vs baseline: 1.0938x; 1.0938x over previous
import jax
import jax.numpy as jnp
import numpy as np
from jax import lax
from jax.experimental import pallas as pl
from jax.experimental.pallas import tpu as pltpu
from jax.experimental.pallas import tpu_sc as plsc

_NUM_USERS = 1000000
_EMB = 128
_LAT = 64
_BATCH = 16384

_SEC = 8192
_MAJ = 4 * _SEC
_NBLK = -(-_NUM_USERS // _MAJ)
_ROWS = _NBLK * _SEC
_LS = _SEC.bit_length() - 1
_LM = _MAJ.bit_length() - 1


def _placement_mats():
    e = np.zeros((8, _LAT, 128), np.float32)
    for k in range(4):
        for w in range(32):
            e[2 * k, w, 32 * k + w] = 1.0
            e[2 * k + 1, 32 + w, 32 * k + w] = 1.0
    return jnp.asarray(e)


def _repack_body(x_ref, e_ref, out_ref):
    dn = (((0,), (0,)), ((), ()))
    acc1 = None
    acc2 = None
    for k in range(4):
        x = x_ref[:, k * _SEC:(k + 1) * _SEC]
        t1 = lax.dot_general(x, e_ref[2 * k], dn,
                             precision=lax.Precision.DEFAULT,
                             preferred_element_type=jnp.float32)
        t2 = lax.dot_general(x, e_ref[2 * k + 1], dn,
                             precision=lax.Precision.DEFAULT,
                             preferred_element_type=jnp.float32)
        acc1 = t1 if acc1 is None else acc1 + t1
        acc2 = t2 if acc2 is None else acc2 + t2
    word = lax.bitcast_convert_type(acc1, jnp.int32) | \
        lax.shift_right_logical(lax.bitcast_convert_type(acc2, jnp.int32), 16)
    out_ref[...] = lax.bitcast_convert_type(word, jnp.float32)


def _repack_table(table_t, emats):
    return pl.pallas_call(
        _repack_body,
        grid=(_NBLK,),
        in_specs=[
            pl.BlockSpec((_LAT, _MAJ), lambda i: (0, i)),
            pl.BlockSpec((8, _LAT, 128), lambda i: (0, 0, 0)),
        ],
        out_specs=pl.BlockSpec((_SEC, 128), lambda i: (i, 0)),
        out_shape=jax.ShapeDtypeStruct((_ROWS, 128), jnp.float32),
        compiler_params=pltpu.CompilerParams(
            vmem_limit_bytes=100 * 1024 * 1024),
    )(table_t, emats)


_NC, _NS = 2, 16
_NW = _NC * _NS
_BPW = _BATCH // _NW
_CHUNK = 128
_NCH = _BPW // _CHUNK


def _gather_body(tp_hbm, idx_hbm, out_hbm, idx_v, pidx_v, g_v, sem):
    wid = lax.axis_index("s") * _NC + lax.axis_index("c")
    base = wid * _BPW
    pltpu.sync_copy(idx_hbm.at[pl.ds(base, _BPW)], idx_v)
    for j in range(_NCH):
        for g in range(_CHUNK // 16):
            s = pl.ds(j * _CHUNK + g * 16, 16)
            u = idx_v[s]
            r = lax.shift_left(lax.shift_right_logical(u, _LM), _LS) | \
                (u & (_SEC - 1))
            pidx_v[j, pl.ds(g * 16, 16)] = r
    copies = [
        pltpu.async_copy(tp_hbm.at[pidx_v.at[j]], g_v.at[j], sem)
        for j in range(_NCH)
    ]
    for j in range(_NCH):
        copies[j].wait()
        pltpu.sync_copy(g_v.at[j],
                        out_hbm.at[pl.ds(base + j * _CHUNK, _CHUNK)])


_sc_gather = pl.kernel(
    _gather_body,
    out_type=jax.ShapeDtypeStruct((_BATCH, 128), jnp.float32),
    mesh=plsc.VectorSubcoreMesh(core_axis_name="c", subcore_axis_name="s"),
    scratch_types=[
        pltpu.VMEM((_BPW,), jnp.int32),
        pltpu.VMEM((_NCH, _CHUNK), jnp.int32),
        pltpu.VMEM((_NCH, _CHUNK, 128), jnp.float32),
        pltpu.SemaphoreType.DMA,
    ],
)

_BLK = 2048


def _combine_body(emb_ref, w2_ref, b2_ref, g_ref, idx_ref, out_ref):
    v2 = jnp.dot(emb_ref[...], w2_ref[...],
                 preferred_element_type=jnp.float32,
                 precision=lax.Precision.DEFAULT) + b2_ref[...]
    vh = v2[:, :128]
    vl = v2[:, 128:]
    gbits = lax.bitcast_convert_type(g_ref[...], jnp.int32)
    u_hi = lax.bitcast_convert_type(gbits & jnp.int32(-65536), jnp.float32)
    u_lo = lax.bitcast_convert_type(gbits << 16, jnp.float32)
    idx = idx_ref[...]
    s = (idx & (_MAJ - 1)) >> _LS
    laneq = lax.broadcasted_iota(jnp.int32, (_BLK, 128), 1) >> 5
    m = jnp.where(laneq == s, 1.0, 0.0)
    prod = (u_hi * vh + u_lo * vl) * m
    ones = jnp.ones((1, 128), jnp.float32)
    srow = lax.dot_general(ones, prod, (((1,), (1,)), ((), ())),
                           precision=lax.Precision.HIGHEST,
                           preferred_element_type=jnp.float32)
    out_ref[...] = srow[None]


def _tc_combine(g_rows, idx2, item_emb, w2, b2):
    out = pl.pallas_call(
        _combine_body,
        grid=(_BATCH // _BLK,),
        in_specs=[
            pl.BlockSpec((_BLK, _EMB), lambda i: (i, 0)),
            pl.BlockSpec((_EMB, 256), lambda i: (0, 0)),
            pl.BlockSpec((1, 256), lambda i: (0, 0)),
            pl.BlockSpec((_BLK, 128), lambda i: (i, 0)),
            pl.BlockSpec((_BLK, 1), lambda i: (i, 0)),
        ],
        out_specs=pl.BlockSpec((1, 1, _BLK), lambda i: (i, 0, 0)),
        out_shape=jax.ShapeDtypeStruct((_BATCH // _BLK, 1, _BLK), jnp.float32),
    )(item_emb, w2, b2, g_rows, idx2)
    return out.reshape(-1)


def kernel(user_idx, item_emb, user_table, W, b):
    idx = user_idx.astype(jnp.int32)
    table_packed = _repack_table(user_table.T, _placement_mats())
    g_rows = _sc_gather(table_packed, idx)
    idx2 = idx.reshape(_BATCH, 1)
    wh = jnp.tile(W[:, :32], (1, 4))
    wl = jnp.tile(W[:, 32:], (1, 4))
    w2 = jnp.concatenate([wh, wl], axis=1)
    b2 = jnp.concatenate([jnp.tile(b[:32], 4),
                          jnp.tile(b[32:], 4)]).reshape(1, 256)
    return _tc_combine(g_rows, idx2, item_emb, w2, b2)


# speedup vs baseline: 1.1034x; 1.0089x over previous
import jax
import jax.numpy as jnp
import numpy as np
from jax import lax
from jax.experimental import pallas as pl
from jax.experimental.pallas import tpu as pltpu
from jax.experimental.pallas import tpu_sc as plsc

_NUM_USERS = 1000000
_EMB = 128
_LAT = 64
_BATCH = 16384

_SEC = 8192
_MAJ = 4 * _SEC
_NBLK = -(-_NUM_USERS // _MAJ)
_ROWS = _NBLK * _SEC
_LS = _SEC.bit_length() - 1
_LM = _MAJ.bit_length() - 1


def _placement_mats():
    e = np.zeros((8, _LAT, 128), np.float32)
    for k in range(4):
        for w in range(32):
            e[2 * k, w, 32 * k + w] = 1.0
            e[2 * k + 1, 32 + w, 32 * k + w] = 1.0
    return jnp.asarray(e)


def _repack_body(x_ref, e_ref, out_ref):
    dn = (((0,), (0,)), ((), ()))
    acc1 = None
    acc2 = None
    for k in range(4):
        x = x_ref[:, k * _SEC:(k + 1) * _SEC]
        t1 = lax.dot_general(x, e_ref[2 * k], dn,
                             precision=lax.Precision.DEFAULT,
                             preferred_element_type=jnp.float32)
        t2 = lax.dot_general(x, e_ref[2 * k + 1], dn,
                             precision=lax.Precision.DEFAULT,
                             preferred_element_type=jnp.float32)
        acc1 = t1 if acc1 is None else acc1 + t1
        acc2 = t2 if acc2 is None else acc2 + t2
    word = lax.bitcast_convert_type(acc1, jnp.int32) | \
        lax.shift_right_logical(lax.bitcast_convert_type(acc2, jnp.int32), 16)
    out_ref[...] = lax.bitcast_convert_type(word, jnp.float32)


def _repack_table(table_t, emats):
    return pl.pallas_call(
        _repack_body,
        grid=(_NBLK,),
        in_specs=[
            pl.BlockSpec((_LAT, _MAJ), lambda i: (0, i)),
            pl.BlockSpec((8, _LAT, 128), lambda i: (0, 0, 0)),
        ],
        out_specs=pl.BlockSpec((_SEC, 128), lambda i: (i, 0)),
        out_shape=jax.ShapeDtypeStruct((_ROWS, 128), jnp.float32),
        compiler_params=pltpu.CompilerParams(
            vmem_limit_bytes=100 * 1024 * 1024),
    )(table_t, emats)


_NC, _NS = 2, 16
_NW = _NC * _NS
_BPW = _BATCH // _NW
_CHUNK = 128
_NCH = _BPW // _CHUNK


def _gather_body(tp_hbm, idx_hbm, out_hbm, idx_v, pidx_v, g_v, sem):
    wid = lax.axis_index("s") * _NC + lax.axis_index("c")
    base = wid * _BPW
    pltpu.sync_copy(idx_hbm.at[pl.ds(base, _BPW)], idx_v)
    for j in range(_NCH):
        for g in range(_CHUNK // 16):
            s = pl.ds(j * _CHUNK + g * 16, 16)
            u = idx_v[s]
            r = lax.shift_left(lax.shift_right_logical(u, _LM), _LS) | \
                (u & (_SEC - 1))
            pidx_v[j, pl.ds(g * 16, 16)] = r
    copies = [
        pltpu.async_copy(tp_hbm.at[pidx_v.at[j]], g_v.at[j], sem)
        for j in range(_NCH)
    ]
    for j in range(_NCH):
        copies[j].wait()
        pltpu.sync_copy(g_v.at[j],
                        out_hbm.at[pl.ds(base + j * _CHUNK, _CHUNK)])


_sc_gather = pl.kernel(
    _gather_body,
    out_type=jax.ShapeDtypeStruct((_BATCH, 128), jnp.float32),
    mesh=plsc.VectorSubcoreMesh(core_axis_name="c", subcore_axis_name="s"),
    scratch_types=[
        pltpu.VMEM((_BPW,), jnp.int32),
        pltpu.VMEM((_NCH, _CHUNK), jnp.int32),
        pltpu.VMEM((_NCH, _CHUNK, 128), jnp.float32),
        pltpu.SemaphoreType.DMA,
    ],
)

_BLK = 4096


def _combine_body(emb_ref, w2_ref, b2_ref, g_ref, idx_ref, out_ref):
    v2 = jnp.dot(emb_ref[...], w2_ref[...],
                 preferred_element_type=jnp.float32,
                 precision=lax.Precision.DEFAULT) + b2_ref[...]
    vh = v2[:, :128]
    vl = v2[:, 128:]
    gbits = lax.bitcast_convert_type(g_ref[...], jnp.int32)
    u_hi = lax.bitcast_convert_type(gbits & jnp.int32(-65536), jnp.float32)
    u_lo = lax.bitcast_convert_type(gbits << 16, jnp.float32)
    idx = idx_ref[...]
    s = (idx & (_MAJ - 1)) >> _LS
    laneq = lax.broadcasted_iota(jnp.int32, (_BLK, 128), 1) >> 5
    m = jnp.where(laneq == s, 1.0, 0.0)
    prod = (u_hi * vh + u_lo * vl) * m
    ones = jnp.ones((1, 128), jnp.float32)
    srow = lax.dot_general(ones, prod, (((1,), (1,)), ((), ())),
                           precision=lax.Precision.HIGHEST,
                           preferred_element_type=jnp.float32)
    out_ref[...] = srow[None]


def _tc_combine(g_rows, idx2, item_emb, w2, b2):
    out = pl.pallas_call(
        _combine_body,
        grid=(_BATCH // _BLK,),
        in_specs=[
            pl.BlockSpec((_BLK, _EMB), lambda i: (i, 0)),
            pl.BlockSpec((_EMB, 256), lambda i: (0, 0)),
            pl.BlockSpec((1, 256), lambda i: (0, 0)),
            pl.BlockSpec((_BLK, 128), lambda i: (i, 0)),
            pl.BlockSpec((_BLK, 1), lambda i: (i, 0)),
        ],
        out_specs=pl.BlockSpec((1, 1, _BLK), lambda i: (i, 0, 0)),
        out_shape=jax.ShapeDtypeStruct((_BATCH // _BLK, 1, _BLK), jnp.float32),
    )(item_emb, w2, b2, g_rows, idx2)
    return out.reshape(-1)


def kernel(user_idx, item_emb, user_table, W, b):
    idx = user_idx.astype(jnp.int32)
    table_packed = _repack_table(user_table.T, _placement_mats())
    g_rows = _sc_gather(table_packed, idx)
    idx2 = idx.reshape(_BATCH, 1)
    wh = jnp.tile(W[:, :32], (1, 4))
    wl = jnp.tile(W[:, 32:], (1, 4))
    w2 = jnp.concatenate([wh, wl], axis=1)
    b2 = jnp.concatenate([jnp.tile(b[:32], 4),
                          jnp.tile(b[32:], 4)]).reshape(1, 256)
    return _tc_combine(g_rows, idx2, item_emb, w2, b2)
